# baseline (speedup 1.0000x reference)
import functools

import jax
import jax.numpy as jnp
from jax import lax
from jax.experimental import pallas as pl
from jax.experimental.pallas import tpu as pltpu
from jax.experimental.pallas import tpu_sc as plsc

_NC = 2
_NS = 16
_NW = _NC * _NS

_S = 200
_B = 4096
_D = 32
_SG = 2
_BG = 16
_SPW = _S // _SG
_BPW = _B // _BG
_J = _BPW // 128


def _embed_sc(X, idx_arr):
    mesh = plsc.VectorSubcoreMesh(core_axis_name="c", subcore_axis_name="s")

    @functools.partial(
        pl.kernel,
        mesh=mesh,
        compiler_params=pltpu.CompilerParams(
            use_tc_tiling_on_sc=True, needs_layout_passes=False),
        out_type=jax.ShapeDtypeStruct((_S, _D, _B), jnp.float32),
        scratch_types=[
            pltpu.VMEM((_J * _SPW, 128), jnp.int32),
            pltpu.VMEM((_J, 128), jnp.int32),
            pltpu.VMEM((_J, 128), jnp.int32),
            pltpu.VMEM((_BPW, 128), jnp.float32),
            pltpu.VMEM((_BPW, 128), jnp.float32),
            pltpu.VMEM((_J, _D, 128), jnp.float32),
            pltpu.VMEM((_J, _D, 128), jnp.float32),
            pltpu.SemaphoreType.DMA,
            pltpu.SemaphoreType.DMA,
            pltpu.SemaphoreType.DMA,
            pltpu.SemaphoreType.DMA,
            pltpu.SemaphoreType.DMA,
        ],
    )
    def k(x_hbm, idx_hbm, out_hbm, idx_all, lidx0, lidx1, lines0, lines1,
          st0, st1, sem_i, sem_g0, sem_g1, sem_o0, sem_o1):
        biota = lax.iota(jnp.int32, 16)
        wid = lax.axis_index("s") * _NC + lax.axis_index("c")
        sg = wid // _BG
        bg = wid % _BG
        s_base = sg * _SPW
        b_base = bg * _BPW

        pltpu.sync_copy(idx_hbm.at[wid], idx_all)

        def prep_lidx(t, lidx):
            for j in range(_J):
                for h in range(8):
                    v = idx_all[_J * t + j, pl.ds(h * 16, 16)]
                    lidx[j, pl.ds(h * 16, 16)] = lax.shift_right_logical(v, 2)

        def fire_streams(lidx, lines, sem):
            cs = []
            for j in range(_J):
                cs.append(pltpu.async_copy(
                    x_hbm.at[lidx.at[j]],
                    lines.at[pl.ds(j * 128, 128)],
                    sem,
                ))
            return cs

        def wait_streams(lines, sem):
            for j in range(_J):
                pltpu.make_async_copy(
                    x_hbm.at[pl.ds(0, 128)],
                    lines.at[pl.ds(j * 128, 128)],
                    sem,
                ).wait()

        def extract(t, lines, st):
            def per_group(g):
                j = g // 8
                rb0 = (g % 8) * 16
                idx16 = idx_all[_J * t + j, pl.ds(rb0, 16)]
                off16 = lax.shift_left(
                    lax.bitwise_and(idx16, jnp.int32(3)), 5)
                rowv = g * 16 + biota
                for d in range(_D):
                    colv = off16 + d
                    vec = plsc.load_gather(lines, [rowv, colv])
                    st[j, d, pl.ds(rb0, 16)] = vec
            plsc.parallel_loop(0, 8 * _J, 1, unroll=4)(per_group)

        def start_out(t, st, sem):
            s = s_base + t
            for j in range(_J):
                pltpu.async_copy(
                    st.at[j],
                    out_hbm.at[s, pl.ds(0, _D), pl.ds(b_base + j * 128, 128)],
                    sem,
                )

        def wait_out(st, sem):
            for j in range(_J):
                pltpu.make_async_copy(
                    st.at[j],
                    out_hbm.at[0, pl.ds(0, _D), pl.ds(b_base + j * 128, 128)],
                    sem,
                ).wait()

        prep_lidx(0, lidx0)
        fire_streams(lidx0, lines0, sem_g0)
        prep_lidx(1, lidx1)
        fire_streams(lidx1, lines1, sem_g1)

        def body(i, carry):
            t0 = 2 * i
            wait_streams(lines0, sem_g0)
            extract(t0, lines0, st0)
            start_out(t0, st0, sem_o0)
            prep_lidx(t0 + 2, lidx0)
            fire_streams(lidx0, lines0, sem_g0)
            wait_streams(lines1, sem_g1)
            extract(t0 + 1, lines1, st1)
            start_out(t0 + 1, st1, sem_o1)
            prep_lidx(t0 + 3, lidx1)
            fire_streams(lidx1, lines1, sem_g1)
            wait_out(st0, sem_o0)
            wait_out(st1, sem_o1)
            return carry

        lax.fori_loop(0, _SPW // 2 - 1, body, 0)

        t_last = _SPW - 2
        wait_streams(lines0, sem_g0)
        extract(t_last, lines0, st0)
        start_out(t_last, st0, sem_o0)
        wait_streams(lines1, sem_g1)
        extract(t_last + 1, lines1, st1)
        start_out(t_last + 1, st1, sem_o1)
        wait_out(st0, sem_o0)
        wait_out(st1, sem_o1)

    return k(X, idx_arr)


def _mask_tc(inputs_t):
    def body(x_ref, o_ref):
        o_ref[...] = x_ref[...] != 0

    return pl.pallas_call(
        body,
        grid=(5,),
        in_specs=[pl.BlockSpec((_S // 5, _B), lambda i: (i, 0))],
        out_specs=pl.BlockSpec((_S // 5, _B), lambda i: (i, 0)),
        out_shape=jax.ShapeDtypeStruct((_S, _B), jnp.bool_),
    )(inputs_t)


def kernel(inputs, table):
    inputs_t = inputs.T
    idx_arr = (inputs_t.reshape(_SG, _SPW, _BG, _BPW)
               .transpose(0, 2, 1, 3)
               .reshape(_NW, _J * _SPW, 128))
    X = table.reshape(25000, 128)
    out3 = _embed_sc(X, idx_arr)
    emb = jnp.transpose(out3, (2, 0, 1))
    mask = _mask_tc(inputs_t).T
    return emb, mask


# speedup vs baseline: 1.1711x; 1.1711x over previous
import functools

import jax
import jax.numpy as jnp
from jax import lax
from jax.experimental import pallas as pl
from jax.experimental.pallas import tpu as pltpu
from jax.experimental.pallas import tpu_sc as plsc

_NC = 2
_NS = 16
_NW = _NC * _NS

_S = 200
_B = 4096
_D = 32
_N = _S * _B
_LINES = _N * _D // 128
_LPW = _LINES // _NW
_CH = 512
_CHL = _CH * _D // 128
_NCH = _N // _NW // _CH


def _gather_sc(table, idx):
    mesh = plsc.VectorSubcoreMesh(core_axis_name="c", subcore_axis_name="s")

    @functools.partial(
        pl.kernel,
        mesh=mesh,
        compiler_params=pltpu.CompilerParams(use_tc_tiling_on_sc=False),
        out_type=jax.ShapeDtypeStruct((_LINES, 128), jnp.float32),
        scratch_types=[
            pltpu.VMEM((_CH // 128 * _NCH, 128), jnp.int32),
            pltpu.VMEM((_CH, _D), jnp.float32),
            pltpu.VMEM((_CH, _D), jnp.float32),
            pltpu.VMEM((_CHL, 128), jnp.float32),
            pltpu.VMEM((_CHL, 128), jnp.float32),
            pltpu.SemaphoreType.DMA,
            pltpu.SemaphoreType.DMA,
            pltpu.SemaphoreType.DMA,
            pltpu.SemaphoreType.DMA,
        ],
    )
    def k(tab_hbm, idx_hbm, out_hbm, idx_all, rows0, rows1, st0, st1,
          sem_g0, sem_g1, sem_o0, sem_o1):
        wid = lax.axis_index("s") * _NC + lax.axis_index("c")
        nrow_w = _CH // 128 * _NCH
        pltpu.sync_copy(idx_hbm.at[pl.ds(wid * nrow_w, nrow_w)], idx_all)
        line_base = wid * _LPW

        def fire(t, rows, sem):
            for j in range(_CH // 128):
                pltpu.async_copy(
                    tab_hbm.at[idx_all.at[t * (_CH // 128) + j]],
                    rows.at[pl.ds(j * 128, 128)],
                    sem,
                )

        def wait_g(rows, sem):
            for j in range(_CH // 128):
                pltpu.make_async_copy(
                    tab_hbm.at[pl.ds(0, 128)],
                    rows.at[pl.ds(j * 128, 128)],
                    sem,
                ).wait()

        def repack(rows, st):
            def per_row(r):
                for c in range(8):
                    st[r, pl.ds(c * 16, 16)] = rows[
                        4 * r + (c // 2), pl.ds((c % 2) * 16, 16)]
            plsc.parallel_loop(0, _CHL, 1, unroll=2)(per_row)

        def start_out(t, st, sem):
            pltpu.async_copy(
                st, out_hbm.at[pl.ds(line_base + t * _CHL, _CHL)], sem)

        def wait_out(st, sem):
            pltpu.make_async_copy(
                st, out_hbm.at[pl.ds(line_base, _CHL)], sem).wait()

        fire(0, rows0, sem_g0)
        fire(1, rows1, sem_g1)

        def body(i, carry):
            t0 = 2 * i
            wait_g(rows0, sem_g0)
            repack(rows0, st0)
            fire(t0 + 2, rows0, sem_g0)
            start_out(t0, st0, sem_o0)
            wait_g(rows1, sem_g1)
            repack(rows1, st1)
            fire(t0 + 3, rows1, sem_g1)
            start_out(t0 + 1, st1, sem_o1)
            wait_out(st0, sem_o0)
            wait_out(st1, sem_o1)
            return carry

        lax.fori_loop(0, _NCH // 2 - 1, body, 0)

        t_last = _NCH - 2
        wait_g(rows0, sem_g0)
        repack(rows0, st0)
        start_out(t_last, st0, sem_o0)
        wait_g(rows1, sem_g1)
        repack(rows1, st1)
        start_out(t_last + 1, st1, sem_o1)
        wait_out(st0, sem_o0)
        wait_out(st1, sem_o1)

    return k(table, idx)


def _format_tc(inter):
    def body(x_ref, o_ref):
        xT = x_ref[...].T
        for kq in range(4):
            o_ref[0, :, pl.ds(kq * (_B // 4), _B // 4)] = (
                xT[kq * _D:(kq + 1) * _D, :])

    return pl.pallas_call(
        body,
        grid=(_S,),
        in_specs=[pl.BlockSpec((_B * _D // 128, 128), lambda i: (i, 0))],
        out_specs=pl.BlockSpec((1, _D, _B), lambda i: (i, 0, 0)),
        out_shape=jax.ShapeDtypeStruct((_S, _D, _B), jnp.float32),
    )(inter)


def _mask_tc(inputs_t):
    def body(x_ref, o_ref):
        o_ref[...] = x_ref[...] != 0

    return pl.pallas_call(
        body,
        grid=(5,),
        in_specs=[pl.BlockSpec((_S // 5, _B), lambda i: (i, 0))],
        out_specs=pl.BlockSpec((_S // 5, _B), lambda i: (i, 0)),
        out_shape=jax.ShapeDtypeStruct((_S, _B), jnp.bool_),
    )(inputs_t)


def kernel(inputs, table):
    inputs_t = inputs.T
    idx = (inputs_t.reshape(_S, 4, _B // 4)
           .transpose(0, 2, 1)
           .reshape(_N // 128, 128))
    inter = _gather_sc(table, idx)
    out3 = _format_tc(inter)
    emb = jnp.transpose(out3, (2, 0, 1))
    mask = _mask_tc(inputs_t).T
    return emb, mask


# speedup vs baseline: 1.6653x; 1.4220x over previous
import functools

import jax
import jax.numpy as jnp
from jax import lax
from jax.experimental import pallas as pl
from jax.experimental.pallas import tpu as pltpu
from jax.experimental.pallas import tpu_sc as plsc

_NC = 2
_NS = 16
_NW = _NC * _NS

_S = 200
_B = 4096
_D = 32
_N = _S * _B
_LINES = _N * _D // 128
_LPW = _LINES // _NW
_CH = 512
_CHL = _CH * _D // 128
_NCH = _N // _NW // _CH


def _gather_sc(table, idx):
    mesh = plsc.VectorSubcoreMesh(core_axis_name="c", subcore_axis_name="s")

    @functools.partial(
        pl.kernel,
        mesh=mesh,
        compiler_params=pltpu.CompilerParams(use_tc_tiling_on_sc=False),
        out_type=jax.ShapeDtypeStruct((_LINES, 128), jnp.float32),
        scratch_types=[
            pltpu.VMEM((224, 128), jnp.int32),
            pltpu.VMEM((_CH, _D), jnp.float32),
            pltpu.VMEM((_CH, _D), jnp.float32),
            pltpu.VMEM((_CHL, 128), jnp.float32),
            pltpu.VMEM((_CHL, 128), jnp.float32),
            pltpu.SemaphoreType.DMA,
            pltpu.SemaphoreType.DMA,
            pltpu.SemaphoreType.DMA,
            pltpu.SemaphoreType.DMA,
        ],
    )
    def k(tab_hbm, idx_hbm, out_hbm, idx_all, rows0, rows1, st0, st1,
          sem_g0, sem_g1, sem_o0, sem_o1):
        wid = lax.axis_index("s") * _NC + lax.axis_index("c")
        line_base = wid * _LPW
        rows_base = (wid * _NCH // 8) * 32
        pltpu.sync_copy(idx_hbm.at[pl.ds(rows_base, 224)], idx_all)

        def fire(t, rows, sem):
            g = wid * _NCH + t
            row0 = (g // 8) * 32 + (g % 8) - rows_base
            for kq in range(4):
                pltpu.async_copy(
                    tab_hbm.at[idx_all.at[row0 + kq * 8]],
                    rows.at[pl.ds(kq * 128, 128)],
                    sem,
                )

        def wait_g(rows, sem):
            for j in range(_CH // 128):
                pltpu.make_async_copy(
                    tab_hbm.at[pl.ds(0, 128)],
                    rows.at[pl.ds(j * 128, 128)],
                    sem,
                ).wait()

        def repack(rows, st):
            def per_row(r):
                for c in range(8):
                    kq, h = c // 2, c % 2
                    st[r, pl.ds(c * 16, 16)] = rows[
                        kq * 128 + r, pl.ds(h * 16, 16)]
            plsc.parallel_loop(0, _CHL, 1, unroll=2)(per_row)

        def start_out(t, st, sem):
            pltpu.async_copy(
                st, out_hbm.at[pl.ds(line_base + t * _CHL, _CHL)], sem)

        def wait_out(st, sem):
            pltpu.make_async_copy(
                st, out_hbm.at[pl.ds(line_base, _CHL)], sem).wait()

        fire(0, rows0, sem_g0)
        fire(1, rows1, sem_g1)

        def body(i, carry):
            t0 = 2 * i
            wait_g(rows0, sem_g0)
            repack(rows0, st0)
            fire(t0 + 2, rows0, sem_g0)
            start_out(t0, st0, sem_o0)
            wait_g(rows1, sem_g1)
            repack(rows1, st1)
            fire(t0 + 3, rows1, sem_g1)
            start_out(t0 + 1, st1, sem_o1)
            wait_out(st0, sem_o0)
            wait_out(st1, sem_o1)
            return carry

        lax.fori_loop(0, _NCH // 2 - 1, body, 0)

        t_last = _NCH - 2
        wait_g(rows0, sem_g0)
        repack(rows0, st0)
        start_out(t_last, st0, sem_o0)
        wait_g(rows1, sem_g1)
        repack(rows1, st1)
        start_out(t_last + 1, st1, sem_o1)
        wait_out(st0, sem_o0)
        wait_out(st1, sem_o1)

    return k(table, idx)


def _format_tc(inter):
    def body(x_ref, o_ref):
        xT = x_ref[...].T
        for kq in range(4):
            o_ref[0, :, pl.ds(kq * (_B // 4), _B // 4)] = (
                xT[kq * _D:(kq + 1) * _D, :])

    return pl.pallas_call(
        body,
        grid=(_S,),
        in_specs=[pl.BlockSpec((_B * _D // 128, 128), lambda i: (i, 0))],
        out_specs=pl.BlockSpec((1, _D, _B), lambda i: (i, 0, 0)),
        out_shape=jax.ShapeDtypeStruct((_S, _D, _B), jnp.float32),
    )(inter)


def _mask_tc(inputs_t):
    def body(x_ref, o_ref):
        o_ref[...] = x_ref[...] != 0

    return pl.pallas_call(
        body,
        grid=(5,),
        in_specs=[pl.BlockSpec((_S // 5, _B), lambda i: (i, 0))],
        out_specs=pl.BlockSpec((_S // 5, _B), lambda i: (i, 0)),
        out_shape=jax.ShapeDtypeStruct((_S, _B), jnp.bool_),
    )(inputs_t)


def kernel(inputs, table):
    inputs_t = inputs.T
    idx = inputs_t.reshape(_N // 128, 128)
    inter = _gather_sc(table, idx)
    out3 = _format_tc(inter)
    emb = jnp.transpose(out3, (2, 0, 1))
    mask = _mask_tc(inputs_t).T
    return emb, mask


# speedup vs baseline: 2.2245x; 1.3358x over previous
import functools

import jax
import jax.numpy as jnp
from jax import lax
from jax.experimental import pallas as pl
from jax.experimental.pallas import tpu as pltpu
from jax.experimental.pallas import tpu_sc as plsc

_NC = 2
_NS = 16
_NW = _NC * _NS

_S = 200
_B = 4096
_D = 32
_N = _S * _B
_LINES = _N * _D // 128
_LPW = _LINES // _NW
_CH = 512
_CHL = _CH * _D // 128
_NCH = _N // _NW // _CH


def _gather_sc(table, idx):
    mesh = plsc.VectorSubcoreMesh(core_axis_name="c", subcore_axis_name="s")

    @functools.partial(
        pl.kernel,
        mesh=mesh,
        compiler_params=pltpu.CompilerParams(use_tc_tiling_on_sc=False),
        out_type=jax.ShapeDtypeStruct((_LINES, 128), jnp.float32),
        scratch_types=[
            pltpu.VMEM((224, 128), jnp.int32),
            pltpu.VMEM((_CH, _D), jnp.float32),
            pltpu.VMEM((_CH, _D), jnp.float32),
            pltpu.VMEM((_CHL, 128), jnp.float32),
            pltpu.VMEM((_CHL, 128), jnp.float32),
            pltpu.SemaphoreType.DMA,
            pltpu.SemaphoreType.DMA,
            pltpu.SemaphoreType.DMA,
            pltpu.SemaphoreType.DMA,
        ],
    )
    def k(tab_hbm, idx_hbm, out_hbm, idx_all, rows0, rows1, st0, st1,
          sem_g0, sem_g1, sem_o0, sem_o1):
        wid = lax.axis_index("s") * _NC + lax.axis_index("c")
        line_base = wid * _LPW
        rows_base = (wid * _NCH // 8) * 32
        pltpu.sync_copy(idx_hbm.at[pl.ds(rows_base, 224)], idx_all)

        def fire(t, rows, sem):
            g = wid * _NCH + t
            row0 = (g // 8) * 32 + (g % 8) - rows_base
            for kq in range(4):
                pltpu.async_copy(
                    tab_hbm.at[idx_all.at[row0 + kq * 8]],
                    rows.at[pl.ds(kq * 128, 128)],
                    sem,
                )

        def wait_g(rows, sem):
            for j in range(_CH // 128):
                pltpu.make_async_copy(
                    tab_hbm.at[pl.ds(0, 128)],
                    rows.at[pl.ds(j * 128, 128)],
                    sem,
                ).wait()

        def repack(rows, st):
            def per_row(r):
                for c in range(8):
                    kq, h = c // 2, c % 2
                    st[r, pl.ds(c * 16, 16)] = rows[
                        kq * 128 + r, pl.ds(h * 16, 16)]
            plsc.parallel_loop(0, _CHL, 1, unroll=2)(per_row)

        def start_out(t, st, sem):
            pltpu.async_copy(
                st, out_hbm.at[pl.ds(line_base + t * _CHL, _CHL)], sem)

        def wait_out(st, sem):
            pltpu.make_async_copy(
                st, out_hbm.at[pl.ds(line_base, _CHL)], sem).wait()

        fire(0, rows0, sem_g0)
        fire(1, rows1, sem_g1)

        def body(i, carry):
            t0 = 2 * i
            wait_g(rows0, sem_g0)
            repack(rows0, st0)
            fire(t0 + 2, rows0, sem_g0)
            start_out(t0, st0, sem_o0)
            wait_g(rows1, sem_g1)
            repack(rows1, st1)
            fire(t0 + 3, rows1, sem_g1)
            start_out(t0 + 1, st1, sem_o1)
            wait_out(st0, sem_o0)
            wait_out(st1, sem_o1)
            return carry

        lax.fori_loop(0, _NCH // 2 - 1, body, 0)

        t_last = _NCH - 2
        wait_g(rows0, sem_g0)
        repack(rows0, st0)
        start_out(t_last, st0, sem_o0)
        wait_g(rows1, sem_g1)
        repack(rows1, st1)
        start_out(t_last + 1, st1, sem_o1)
        wait_out(st0, sem_o0)
        wait_out(st1, sem_o1)

    return k(table, idx)


def _format_tc(inter):
    _SB = 4

    def body(x_ref, o_ref):
        for si in range(_SB):
            xT = x_ref[pl.ds(si * (_B * _D // 128), _B * _D // 128), :].T
            for kq in range(4):
                o_ref[si, :, pl.ds(kq * (_B // 4), _B // 4)] = (
                    xT[kq * _D:(kq + 1) * _D, :])

    return pl.pallas_call(
        body,
        grid=(_S // _SB,),
        in_specs=[pl.BlockSpec((_SB * _B * _D // 128, 128), lambda i: (i, 0))],
        out_specs=pl.BlockSpec((_SB, _D, _B), lambda i: (i, 0, 0)),
        out_shape=jax.ShapeDtypeStruct((_S, _D, _B), jnp.float32),
    )(inter)


def _mask_tc(inputs_t):
    def body(x_ref, o_ref):
        o_ref[...] = x_ref[...] != 0

    return pl.pallas_call(
        body,
        grid=(5,),
        in_specs=[pl.BlockSpec((_S // 5, _B), lambda i: (i, 0))],
        out_specs=pl.BlockSpec((_S // 5, _B), lambda i: (i, 0)),
        out_shape=jax.ShapeDtypeStruct((_S, _B), jnp.bool_),
    )(inputs_t)


def kernel(inputs, table):
    inputs_t = inputs.T
    idx = inputs_t.reshape(_N // 128, 128)
    inter = _gather_sc(table, idx)
    out3 = _format_tc(inter)
    emb = jnp.transpose(out3, (2, 0, 1))
    mask = _mask_tc(inputs_t).T
    return emb, mask


# speedup vs baseline: 2.3362x; 1.0503x over previous
import functools

import jax
import jax.numpy as jnp
from jax import lax
from jax.experimental import pallas as pl
from jax.experimental.pallas import tpu as pltpu
from jax.experimental.pallas import tpu_sc as plsc

_NC = 2
_NS = 16
_NW = _NC * _NS

_S = 200
_B = 4096
_D = 32
_N = _S * _B
_LINES = _N * _D // 128
_LPW = _LINES // _NW
_CH = 512
_CHL = _CH * _D // 128
_NCH = _N // _NW // _CH


def _gather_sc(table, idx):
    mesh = plsc.VectorSubcoreMesh(core_axis_name="c", subcore_axis_name="s")

    @functools.partial(
        pl.kernel,
        mesh=mesh,
        compiler_params=pltpu.CompilerParams(use_tc_tiling_on_sc=False),
        out_type=jax.ShapeDtypeStruct((_LINES, 128), jnp.float32),
        scratch_types=[
            pltpu.VMEM((224, 128), jnp.int32),
            pltpu.VMEM((_CH, _D), jnp.float32),
            pltpu.VMEM((_CH, _D), jnp.float32),
            pltpu.VMEM((_CHL, 128), jnp.float32),
            pltpu.VMEM((_CHL, 128), jnp.float32),
            pltpu.SemaphoreType.DMA,
            pltpu.SemaphoreType.DMA,
            pltpu.SemaphoreType.DMA,
            pltpu.SemaphoreType.DMA,
        ],
    )
    def k(tab_hbm, idx_hbm, out_hbm, idx_all, rows0, rows1, st0, st1,
          sem_g0, sem_g1, sem_o0, sem_o1):
        wid = lax.axis_index("s") * _NC + lax.axis_index("c")
        line_base = wid * _LPW
        rows_base = (wid * _NCH // 8) * 32
        pltpu.sync_copy(idx_hbm.at[pl.ds(rows_base, 224)], idx_all)

        def fire(t, rows, sem):
            g = wid * _NCH + t
            row0 = (g // 8) * 32 + (g % 8) - rows_base
            for kq in range(4):
                pltpu.async_copy(
                    tab_hbm.at[idx_all.at[row0 + kq * 8]],
                    rows.at[pl.ds(kq * 128, 128)],
                    sem,
                )

        def wait_g(rows, sem):
            for j in range(_CH // 128):
                pltpu.make_async_copy(
                    tab_hbm.at[pl.ds(0, 128)],
                    rows.at[pl.ds(j * 128, 128)],
                    sem,
                ).wait()

        def repack(rows, st):
            def per_row(r):
                for c in range(8):
                    kq, h = c // 2, c % 2
                    st[r, pl.ds(c * 16, 16)] = rows[
                        kq * 128 + r, pl.ds(h * 16, 16)]
            plsc.parallel_loop(0, _CHL, 1, unroll=2)(per_row)

        def start_out(t, st, sem):
            pltpu.async_copy(
                st, out_hbm.at[pl.ds(line_base + t * _CHL, _CHL)], sem)

        def wait_out(st, sem):
            pltpu.make_async_copy(
                st, out_hbm.at[pl.ds(line_base, _CHL)], sem).wait()

        fire(0, rows0, sem_g0)
        fire(1, rows1, sem_g1)

        def body(i, carry):
            t0 = 2 * i
            wait_g(rows0, sem_g0)
            repack(rows0, st0)
            fire(t0 + 2, rows0, sem_g0)
            start_out(t0, st0, sem_o0)
            wait_g(rows1, sem_g1)
            repack(rows1, st1)
            fire(t0 + 3, rows1, sem_g1)
            start_out(t0 + 1, st1, sem_o1)
            wait_out(st0, sem_o0)
            wait_out(st1, sem_o1)
            return carry

        lax.fori_loop(0, _NCH // 2 - 1, body, 0)

        t_last = _NCH - 2
        wait_g(rows0, sem_g0)
        repack(rows0, st0)
        start_out(t_last, st0, sem_o0)
        wait_g(rows1, sem_g1)
        repack(rows1, st1)
        start_out(t_last + 1, st1, sem_o1)
        wait_out(st0, sem_o0)
        wait_out(st1, sem_o1)

    return k(table, idx)


def _format_tc(inter):
    _SB = 8

    def body(x_ref, o_ref):
        for si in range(_SB):
            xT = x_ref[pl.ds(si * (_B * _D // 128), _B * _D // 128), :].T
            for kq in range(4):
                o_ref[si, :, pl.ds(kq * (_B // 4), _B // 4)] = (
                    xT[kq * _D:(kq + 1) * _D, :])

    return pl.pallas_call(
        body,
        grid=(_S // _SB,),
        in_specs=[pl.BlockSpec((_SB * _B * _D // 128, 128), lambda i: (i, 0))],
        out_specs=pl.BlockSpec((_SB, _D, _B), lambda i: (i, 0, 0)),
        out_shape=jax.ShapeDtypeStruct((_S, _D, _B), jnp.float32),
    )(inter)


def _mask_tc(inputs_t):
    def body(x_ref, o_ref):
        o_ref[...] = x_ref[...] != 0

    return pl.pallas_call(
        body,
        grid=(5,),
        in_specs=[pl.BlockSpec((_S // 5, _B), lambda i: (i, 0))],
        out_specs=pl.BlockSpec((_S // 5, _B), lambda i: (i, 0)),
        out_shape=jax.ShapeDtypeStruct((_S, _B), jnp.bool_),
    )(inputs_t)


def kernel(inputs, table):
    inputs_t = inputs.T
    idx = inputs_t.reshape(_N // 128, 128)
    inter = _gather_sc(table, idx)
    out3 = _format_tc(inter)
    emb = jnp.transpose(out3, (2, 0, 1))
    mask = _mask_tc(inputs_t).T
    return emb, mask


# speedup vs baseline: 2.3589x; 1.0097x over previous
import functools

import jax
import jax.numpy as jnp
from jax import lax
from jax.experimental import pallas as pl
from jax.experimental.pallas import tpu as pltpu
from jax.experimental.pallas import tpu_sc as plsc

_NC = 2
_NS = 16
_NW = _NC * _NS

_S = 200
_B = 4096
_D = 32
_N = _S * _B
_LINES = _N * _D // 128
_LPW = _LINES // _NW
_CH = 512
_CHL = _CH * _D // 128
_NCH = _N // _NW // _CH


def _gather_sc(table, idx):
    mesh = plsc.VectorSubcoreMesh(core_axis_name="c", subcore_axis_name="s")

    @functools.partial(
        pl.kernel,
        mesh=mesh,
        compiler_params=pltpu.CompilerParams(use_tc_tiling_on_sc=False),
        out_type=jax.ShapeDtypeStruct((_LINES, 128), jnp.float32),
        scratch_types=[
            pltpu.VMEM((224, 128), jnp.int32),
            pltpu.VMEM((_CH, _D), jnp.float32),
            pltpu.VMEM((_CH, _D), jnp.float32),
            pltpu.VMEM((_CHL, 128), jnp.float32),
            pltpu.VMEM((_CHL, 128), jnp.float32),
            pltpu.SemaphoreType.DMA,
            pltpu.SemaphoreType.DMA,
            pltpu.SemaphoreType.DMA,
            pltpu.SemaphoreType.DMA,
        ],
    )
    def k(tab_hbm, idx_hbm, out_hbm, idx_all, rows0, rows1, st0, st1,
          sem_g0, sem_g1, sem_o0, sem_o1):
        wid = lax.axis_index("s") * _NC + lax.axis_index("c")
        line_base = wid * _LPW
        rows_base = (wid * _NCH // 8) * 32
        pltpu.sync_copy(idx_hbm.at[pl.ds(rows_base, 224)], idx_all)

        def fire(t, rows, sem):
            g = wid * _NCH + t
            row0 = (g // 8) * 32 + (g % 8) - rows_base
            for kq in range(4):
                pltpu.async_copy(
                    tab_hbm.at[idx_all.at[row0 + kq * 8]],
                    rows.at[pl.ds(kq * 128, 128)],
                    sem,
                )

        def wait_g(rows, sem):
            for j in range(_CH // 128):
                pltpu.make_async_copy(
                    tab_hbm.at[pl.ds(0, 128)],
                    rows.at[pl.ds(j * 128, 128)],
                    sem,
                ).wait()

        def repack(rows, st):
            def per_row(r):
                for c in range(8):
                    kq, h = c // 2, c % 2
                    st[r, pl.ds(c * 16, 16)] = rows[
                        kq * 128 + r, pl.ds(h * 16, 16)]
            plsc.parallel_loop(0, _CHL, 1, unroll=2)(per_row)

        def start_out(t, st, sem):
            pltpu.async_copy(
                st, out_hbm.at[pl.ds(line_base + t * _CHL, _CHL)], sem)

        def wait_out(st, sem):
            pltpu.make_async_copy(
                st, out_hbm.at[pl.ds(line_base, _CHL)], sem).wait()

        fire(0, rows0, sem_g0)
        fire(1, rows1, sem_g1)

        def body(i, carry):
            t0 = 2 * i
            wait_g(rows0, sem_g0)
            repack(rows0, st0)
            fire(t0 + 2, rows0, sem_g0)
            start_out(t0, st0, sem_o0)
            wait_g(rows1, sem_g1)
            repack(rows1, st1)
            fire(t0 + 3, rows1, sem_g1)
            start_out(t0 + 1, st1, sem_o1)
            wait_out(st0, sem_o0)
            wait_out(st1, sem_o1)
            return carry

        lax.fori_loop(0, _NCH // 2 - 1, body, 0)

        t_last = _NCH - 2
        wait_g(rows0, sem_g0)
        repack(rows0, st0)
        start_out(t_last, st0, sem_o0)
        wait_g(rows1, sem_g1)
        repack(rows1, st1)
        start_out(t_last + 1, st1, sem_o1)
        wait_out(st0, sem_o0)
        wait_out(st1, sem_o1)

    return k(table, idx)


def _format_tc(inter):
    _SB = 20

    def body(x_ref, o_ref):
        for si in range(_SB):
            xT = x_ref[pl.ds(si * (_B * _D // 128), _B * _D // 128), :].T
            for kq in range(4):
                o_ref[si, :, pl.ds(kq * (_B // 4), _B // 4)] = (
                    xT[kq * _D:(kq + 1) * _D, :])

    return pl.pallas_call(
        body,
        grid=(_S // _SB,),
        in_specs=[pl.BlockSpec((_SB * _B * _D // 128, 128), lambda i: (i, 0))],
        out_specs=pl.BlockSpec((_SB, _D, _B), lambda i: (i, 0, 0)),
        out_shape=jax.ShapeDtypeStruct((_S, _D, _B), jnp.float32),
    )(inter)


def _mask_tc(inputs_t):
    def body(x_ref, o_ref):
        o_ref[...] = x_ref[...] != 0

    return pl.pallas_call(
        body,
        grid=(5,),
        in_specs=[pl.BlockSpec((_S // 5, _B), lambda i: (i, 0))],
        out_specs=pl.BlockSpec((_S // 5, _B), lambda i: (i, 0)),
        out_shape=jax.ShapeDtypeStruct((_S, _B), jnp.bool_),
    )(inputs_t)


def kernel(inputs, table):
    inputs_t = inputs.T
    idx = inputs_t.reshape(_N // 128, 128)
    inter = _gather_sc(table, idx)
    out3 = _format_tc(inter)
    emb = jnp.transpose(out3, (2, 0, 1))
    mask = _mask_tc(inputs_t).T
    return emb, mask
